# Initial kernel scaffold; baseline (speedup 1.0000x reference)
import jax
import jax.numpy as jnp
from jax.experimental import pallas as pl


def kernel(input_features, neighbor_inds, guidance, weightnet):
    raise NotImplementedError("write your pallas kernel here")


# speedup vs baseline: 8.4082x; 8.4082x over previous
import functools

import jax
import jax.numpy as jnp
from jax import lax
from jax.experimental import pallas as pl
from jax.experimental.pallas import tpu as pltpu
from jax.experimental.pallas import tpu_sc as plsc

N = 100000
C = 32
K = 16
H = 8
M = 16
L = 16
NC = 2
NS = 16
NW = NC * NS

P = 25
PTS_PER_W = N // NW
CHUNKS = PTS_PER_W // P
IDX_PER_CHUNK = P * K


def _pcf_sc(feat_hbm, inds_hbm, guid_hbm, wn_hbm, out_hbm,
            idx_v, gath_v, guid_v, wn_v, out_v, sem):
    wid = lax.axis_index("s") * NC + lax.axis_index("c")

    base_exp = lax.iota(jnp.int32, L) >> 2
    exp = [base_exp + 4 * j for j in range(4)]
    cm = [jnp.full((L,), c, jnp.int32) for c in range(L)]

    def chunk_body(ch, carry):
        pbase = wid * PTS_PER_W + ch * P

        pltpu.sync_copy(inds_hbm.at[pl.ds(pbase * K, IDX_PER_CHUNK)], idx_v)
        pltpu.sync_copy(guid_hbm.at[pl.ds(pbase * K * H, P * K * H)], guid_v)
        pltpu.sync_copy(wn_hbm.at[pl.ds(pbase * K, IDX_PER_CHUNK)], wn_v)

        cp0 = pltpu.async_copy(
            feat_hbm.at[idx_v.at[pl.ds(0, 128)]], gath_v.at[pl.ds(0, 128)], sem)
        cp1 = pltpu.async_copy(
            feat_hbm.at[idx_v.at[pl.ds(128, 128)]], gath_v.at[pl.ds(128, 128)],
            sem)
        cp2 = pltpu.async_copy(
            feat_hbm.at[idx_v.at[pl.ds(256, 128)]], gath_v.at[pl.ds(256, 128)],
            sem)
        cp3 = pltpu.async_copy(
            feat_hbm.at[idx_v.at[pl.ds(384, 16)]], gath_v.at[pl.ds(384, 16)],
            sem)
        cp0.wait()
        cp1.wait()
        cp2.wait()
        cp3.wait()

        def point_body(p, carry2):
            row0 = p * K
            gbase = p * K * H
            acc = [jnp.zeros((L,), jnp.float32) for _ in range(C)]
            for k in range(0, K, 2):
                gv = guid_v[pl.ds(gbase + k * H, 16)]
                for dk in range(2):
                    row = row0 + k + dk
                    g0 = gv.at[exp[2 * dk]].get(mode="promise_in_bounds")
                    g1 = gv.at[exp[2 * dk + 1]].get(mode="promise_in_bounds")
                    gu0 = gath_v[row, pl.ds(0, L)] * g0
                    gu1 = gath_v[row, pl.ds(L, L)] * g1
                    wrow = wn_v[row, pl.ds(0, M)]
                    for c in range(L):
                        b0 = gu0.at[cm[c]].get(mode="promise_in_bounds")
                        b1 = gu1.at[cm[c]].get(mode="promise_in_bounds")
                        acc[c] = acc[c] + b0 * wrow
                        acc[L + c] = acc[L + c] + b1 * wrow
            obase = p * C * M
            for c in range(C):
                out_v[pl.ds(obase + c * M, M)] = acc[c]
            return carry2

        lax.fori_loop(0, P, point_body, 0)
        pltpu.sync_copy(out_v, out_hbm.at[pl.ds(pbase * C * M, P * C * M)])
        return carry

    lax.fori_loop(0, CHUNKS, chunk_body, 0)


@jax.jit
def _pcf(feat, inds, guid, wn):
    mesh = plsc.VectorSubcoreMesh(core_axis_name="c", subcore_axis_name="s")
    f = functools.partial(
        pl.kernel,
        mesh=mesh,
        compiler_params=pltpu.CompilerParams(use_tc_tiling_on_sc=False),
        out_type=jax.ShapeDtypeStruct((N * C * M,), jnp.float32),
        scratch_types=[
            pltpu.VMEM((IDX_PER_CHUNK,), jnp.int32),
            pltpu.VMEM((IDX_PER_CHUNK, C), jnp.float32),
            pltpu.VMEM((P * K * H,), jnp.float32),
            pltpu.VMEM((IDX_PER_CHUNK, M), jnp.float32),
            pltpu.VMEM((P * C * M,), jnp.float32),
            pltpu.SemaphoreType.DMA,
        ],
    )(_pcf_sc)
    return f(feat, inds, guid, wn)


def kernel(input_features, neighbor_inds, guidance, weightnet):
    b, n, c = input_features.shape
    k = neighbor_inds.shape[2]
    m = weightnet.shape[3]
    feat = input_features[0]
    inds = neighbor_inds[0].astype(jnp.int32).reshape(n * k)
    guid = guidance[0].reshape(-1)
    wn = weightnet[0].reshape(n * k, m)
    out = _pcf(feat, inds, guid, wn)
    return out.reshape(b, n, c * m)


# speedup vs baseline: 10.9429x; 1.3015x over previous
import functools

import jax
import jax.numpy as jnp
from jax import lax
from jax.experimental import pallas as pl
from jax.experimental.pallas import tpu as pltpu
from jax.experimental.pallas import tpu_sc as plsc

N = 100000
C = 32
K = 16
H = 8
M = 16
L = 16
NC = 2
NS = 16
NW = NC * NS

P = 25
PTS_PER_W = N // NW
CHUNKS = PTS_PER_W // P
PK = P * K
GSZ = P * K * H
WSZ = P * K * M
OSZ = P * C * M


def _pcf_sc(feat_hbm, inds_hbm, guid_hbm, wn_hbm, out_hbm,
            idx_v, gath_v, guid_v, wn_v, out_v,
            sem_idx, sem_in, sem_gath, sem_out):
    wid = lax.axis_index("s") * NC + lax.axis_index("c")
    pb0 = wid * PTS_PER_W

    base_exp = lax.iota(jnp.int32, L) >> 2
    exp = [base_exp + 4 * j for j in range(4)]

    def fire_idx(ch, b):
        pltpu.async_copy(inds_hbm.at[pl.ds((pb0 + ch * P) * K, PK)],
                         idx_v.at[pl.ds(b * PK, PK)], sem_idx.at[b])

    def wait_idx(b):
        pltpu.make_async_copy(inds_hbm.at[pl.ds(0, PK)],
                              idx_v.at[pl.ds(b * PK, PK)], sem_idx.at[b]).wait()

    def fire_in(ch, b):
        pbase = pb0 + ch * P
        pltpu.async_copy(guid_hbm.at[pl.ds(pbase * K * H, GSZ)],
                         guid_v.at[pl.ds(b * GSZ, GSZ)], sem_in.at[b])
        pltpu.async_copy(wn_hbm.at[pl.ds(pbase * K * M, WSZ)],
                         wn_v.at[pl.ds(b * WSZ, WSZ)], sem_in.at[b])

    def wait_in(b):
        pltpu.make_async_copy(guid_hbm.at[pl.ds(0, GSZ)],
                              guid_v.at[pl.ds(b * GSZ, GSZ)], sem_in.at[b]).wait()
        pltpu.make_async_copy(wn_hbm.at[pl.ds(0, WSZ)],
                              wn_v.at[pl.ds(b * WSZ, WSZ)], sem_in.at[b]).wait()

    def fire_gather(b):
        for lo, sz in ((0, 128), (128, 128), (256, 128), (384, 16)):
            pltpu.async_copy(
                feat_hbm.at[idx_v.at[pl.ds(b * PK + lo, sz)]],
                gath_v.at[pl.ds(b * PK + lo, sz)], sem_gath.at[b])

    def wait_gather(b):
        for lo, sz in ((0, 128), (128, 128), (256, 128), (384, 16)):
            pltpu.make_async_copy(
                feat_hbm.at[idx_v.at[pl.ds(b * PK + lo, sz)]],
                gath_v.at[pl.ds(b * PK + lo, sz)], sem_gath.at[b]).wait()

    def fire_out(ch, b):
        pltpu.async_copy(out_v.at[pl.ds(b * OSZ, OSZ)],
                         out_hbm.at[pl.ds((pb0 + ch * P) * C * M, OSZ)],
                         sem_out.at[b])

    def wait_out(b):
        pltpu.make_async_copy(out_v.at[pl.ds(b * OSZ, OSZ)],
                              out_hbm.at[pl.ds(0, OSZ)], sem_out.at[b]).wait()

    fire_idx(0, 0)
    fire_in(0, 0)
    wait_idx(0)
    fire_gather(0)
    fire_idx(1, 1)

    def chunk_body(ch, carry):
        b = ch & 1
        nb = 1 - b

        @pl.when(ch + 1 < CHUNKS)
        def _():
            wait_idx(nb)
            fire_gather(nb)
            fire_in(ch + 1, nb)

        wait_gather(b)
        wait_in(b)

        @pl.when(ch + 2 < CHUNKS)
        def _():
            fire_idx(ch + 2, b)

        @pl.when(ch >= 2)
        def _():
            wait_out(b)

        roff = b * PK
        goff = b * GSZ
        woff = b * WSZ
        ooff = b * OSZ

        def point_body(p, carry2):
            row0 = roff + p * K
            gbase = goff + p * K * H
            acc = [jnp.zeros((L,), jnp.float32) for _ in range(C)]
            for k in range(0, K, 2):
                gv = guid_v[pl.ds(gbase + k * H, 16)]
                for dk in range(2):
                    row = row0 + k + dk
                    g0 = gv.at[exp[2 * dk]].get(mode="promise_in_bounds")
                    g1 = gv.at[exp[2 * dk + 1]].get(mode="promise_in_bounds")
                    gu0 = gath_v[row, pl.ds(0, L)] * g0
                    gu1 = gath_v[row, pl.ds(L, L)] * g1
                    wrow = wn_v[pl.ds((row - roff) * M + woff, M)]
                    for c in range(L):
                        acc[c] = acc[c] + gu0[c] * wrow
                        acc[L + c] = acc[L + c] + gu1[c] * wrow
            obase = ooff + p * C * M
            for c in range(C):
                out_v[pl.ds(obase + c * M, M)] = acc[c]
            return carry2

        lax.fori_loop(0, P, point_body, 0)
        fire_out(ch, b)
        return carry

    lax.fori_loop(0, CHUNKS, chunk_body, 0)

    wait_out((CHUNKS - 2) & 1)
    wait_out((CHUNKS - 1) & 1)


@jax.jit
def _pcf(feat, inds, guid, wn):
    mesh = plsc.VectorSubcoreMesh(core_axis_name="c", subcore_axis_name="s")
    f = functools.partial(
        pl.kernel,
        mesh=mesh,
        compiler_params=pltpu.CompilerParams(use_tc_tiling_on_sc=False),
        out_type=jax.ShapeDtypeStruct((N * C * M,), jnp.float32),
        scratch_types=[
            pltpu.VMEM((2 * PK,), jnp.int32),
            pltpu.VMEM((2 * PK, C), jnp.float32),
            pltpu.VMEM((2 * GSZ,), jnp.float32),
            pltpu.VMEM((2 * WSZ,), jnp.float32),
            pltpu.VMEM((2 * OSZ,), jnp.float32),
            pltpu.SemaphoreType.DMA((2,)),
            pltpu.SemaphoreType.DMA((2,)),
            pltpu.SemaphoreType.DMA((2,)),
            pltpu.SemaphoreType.DMA((2,)),
        ],
    )(_pcf_sc)
    return f(feat, inds, guid, wn)


def kernel(input_features, neighbor_inds, guidance, weightnet):
    b, n, c = input_features.shape
    k = neighbor_inds.shape[2]
    m = weightnet.shape[3]
    feat = input_features[0]
    inds = neighbor_inds[0].astype(jnp.int32).reshape(n * k)
    guid = guidance[0].reshape(-1)
    wn = weightnet[0].reshape(-1)
    out = _pcf(feat, inds, guid, wn)
    return out.reshape(b, n, c * m)
